# baseline (speedup 1.0000x reference)
import functools

import jax
import jax.numpy as jnp
from jax.experimental import pallas as pl
from jax.experimental.pallas import tpu as pltpu
from jax.experimental.pallas import tpu_sc as plsc

D_MODEL = 768
D_FF = 3072
N_EXP = 8
TOPK = 2
N_HEADS = 12
EPS = 1e-5
F_BLK = 768


def _ln(x, g, b):
    m = x.mean(-1, keepdims=True)
    v = ((x - m) ** 2).mean(-1, keepdims=True)
    return (x - m) / jnp.sqrt(v + EPS) * g + b


def _mha(x, lp):
    Bq, T, D = x.shape
    H = N_HEADS
    hd = D // H
    q = (x @ lp['wq'] + lp['bq']).reshape(Bq, T, H, hd).transpose(0, 2, 1, 3)
    k = (x @ lp['wk'] + lp['bk']).reshape(Bq, T, H, hd).transpose(0, 2, 1, 3)
    v = (x @ lp['wv'] + lp['bv']).reshape(Bq, T, H, hd).transpose(0, 2, 1, 3)
    s = jnp.einsum('bhtd,bhsd->bhts', q, k) / jnp.sqrt(jnp.float32(hd))
    a = jax.nn.softmax(s, axis=-1)
    o = jnp.einsum('bhts,bhsd->bhtd', a, v).transpose(0, 2, 1, 3).reshape(Bq, T, D)
    return o @ lp['wo'] + lp['bo']


R_BLK = 128


def _gmm_body(bids_ref, eids_ref, valids_ref, offs_ref,
              xs_ref, gs_ref, w1_ref, b1_ref, w2_ref, b2_ref, out_ref):
    s = pl.program_id(0)
    bid = bids_ref[s]
    eid = eids_ref[s]
    prev_bid = bids_ref[jnp.maximum(s - 1, 0)]
    first = (s == 0) | (bid != prev_bid)

    @pl.when(first)
    def _init():
        out_ref[...] = jnp.zeros_like(out_ref)

    @pl.when(valids_ref[s] > 0)
    def _compute():
        rows = bid * R_BLK + jax.lax.broadcasted_iota(jnp.int32, (R_BLK, 1), 0)
        mask = (rows >= offs_ref[eid]) & (rows < offs_ref[eid + 1])
        x = xs_ref[...]
        h = jnp.dot(x, w1_ref[0], preferred_element_type=jnp.float32)
        h = h + b1_ref[0, 0]
        h = 0.5 * h * (1.0 + jax.lax.erf(h * 0.7071067811865476))
        o = jnp.dot(h, w2_ref[0], preferred_element_type=jnp.float32)
        o = (o + b2_ref[0, 0]) * gs_ref[...]
        out_ref[...] += jnp.where(mask, o, 0.0)


def _routing_body(lg_ref, pos_ref, gf_ref, offs_ref, eids_ref, bids_ref, valids_ref):
    T = lg_ref.shape[0]
    P = TOPK * T
    NB = P // R_BLK
    G = NB + N_EXP - 1
    l = lg_ref[...]
    io8 = jax.lax.broadcasted_iota(jnp.int32, (T, N_EXP), 1)
    m1 = jnp.max(l, axis=1, keepdims=True)
    i1 = jnp.min(jnp.where(l == m1, io8, N_EXP), axis=1, keepdims=True)
    lm = jnp.where(io8 == i1, -jnp.inf, l)
    m2 = jnp.max(lm, axis=1, keepdims=True)
    i2 = jnp.min(jnp.where(lm == m2, io8, N_EXP), axis=1, keepdims=True)
    g0 = jax.nn.sigmoid(m1 - m2)
    gf_ref[...] = jnp.concatenate([g0, 1.0 - g0], axis=0)

    e_sel = jnp.concatenate([i1, i2], axis=0)
    iop8 = jax.lax.broadcasted_iota(jnp.int32, (P, N_EXP), 1)
    oh = (iop8 == e_sel).astype(jnp.float32)
    c = oh
    k = 1
    while k < P:
        c = c + jnp.concatenate(
            [jnp.zeros((k, N_EXP), jnp.float32), c[:-k]], axis=0)
        k *= 2
    counts = c[-1:]
    off_x = jax.lax.broadcasted_iota(jnp.int32, (N_EXP + 1, N_EXP), 0)
    off_e = jax.lax.broadcasted_iota(jnp.int32, (N_EXP + 1, N_EXP), 1)
    offs = jnp.sum(jnp.where(off_e < off_x, counts, 0.0), axis=1,
                   keepdims=True)
    offs_ref[...] = offs.astype(jnp.int32)
    t = counts
    k = 1
    while k < N_EXP:
        t = t + jnp.concatenate(
            [jnp.zeros((1, k), jnp.float32), t[:, :-k]], axis=1)
        k *= 2
    offs_row = t - counts
    pos = jnp.sum(oh * (offs_row + c - 1.0), axis=1, keepdims=True)
    pos_ref[...] = pos.astype(jnp.int32)

    offs_e_incl = offs_row + counts
    first = jnp.floor(offs_row / R_BLK)
    last = jnp.floor(jnp.maximum(offs_e_incl - 1.0, 0.0) / R_BLK)
    nvis = jnp.where(counts > 0, last - first + 1.0, 0.0)
    ioge = jax.lax.broadcasted_iota(jnp.int32, (G, N_EXP), 1)
    cumv = nvis
    k = 1
    while k < N_EXP:
        cumv = cumv + jnp.concatenate(
            [jnp.zeros((1, k), jnp.float32), cumv[:, :-k]], axis=1)
        k *= 2
    sg = jax.lax.broadcasted_iota(jnp.int32, (G, 1), 0).astype(jnp.float32)
    eids = jnp.sum((sg >= cumv).astype(jnp.int32), axis=1, keepdims=True)
    eids = jnp.clip(eids, 0, N_EXP - 1)
    valids = (sg < cumv[0, N_EXP - 1]).astype(jnp.int32)
    ohe = (ioge == eids).astype(jnp.float32)
    vstart = jnp.sum(ohe * (cumv - nvis), axis=1, keepdims=True)
    firsts = jnp.sum(ohe * first, axis=1, keepdims=True)
    bids = jnp.clip(firsts + sg - vstart, 0.0, NB - 1.0)
    eids_ref[...] = eids
    bids_ref[...] = bids.astype(jnp.int32)
    valids_ref[...] = valids


def _routing_plan(logits):
    T, E = logits.shape
    P = TOPK * T
    NB = P // R_BLK
    G = NB + N_EXP - 1
    i32 = jnp.int32
    return pl.pallas_call(
        _routing_body,
        out_shape=(jax.ShapeDtypeStruct((P, 1), i32),
                   jax.ShapeDtypeStruct((P, 1), jnp.float32),
                   jax.ShapeDtypeStruct((E + 1, 1), i32),
                   jax.ShapeDtypeStruct((G, 1), i32),
                   jax.ShapeDtypeStruct((G, 1), i32),
                   jax.ShapeDtypeStruct((G, 1), i32)),
    )(logits)


_NW = 32


def _sc_mesh():
    return plsc.VectorSubcoreMesh(core_axis_name="c", subcore_axis_name="s")


def _sc_gather(table, idx):
    B = idx.shape[0]
    D = table.shape[1]
    bpw = B // _NW

    @functools.partial(
        pl.kernel, mesh=_sc_mesh(),
        out_type=jax.ShapeDtypeStruct((B, D), table.dtype),
        scratch_types=[pltpu.VMEM((bpw,), jnp.int32),
                       pltpu.VMEM((bpw, D), jnp.float32),
                       pltpu.SemaphoreType.DMA],
    )
    def k(table_hbm, idx_hbm, out_hbm, idx_v, rows_v, sem):
        wid = jax.lax.axis_index("s") * 2 + jax.lax.axis_index("c")
        base = wid * bpw
        pltpu.sync_copy(idx_hbm.at[pl.ds(base, bpw)], idx_v)
        pltpu.async_copy(table_hbm.at[idx_v], rows_v, sem).wait()
        pltpu.sync_copy(rows_v, out_hbm.at[pl.ds(base, bpw)])

    return k(table, idx)


def _sc_scatter_rows(x, pos):
    T, D = x.shape
    P = pos.shape[0]
    bpw = P // _NW

    @functools.partial(
        pl.kernel, mesh=_sc_mesh(),
        out_type=jax.ShapeDtypeStruct((P, D), x.dtype),
        scratch_types=[pltpu.VMEM((bpw,), jnp.int32),
                       pltpu.VMEM((bpw, D), jnp.float32),
                       pltpu.SemaphoreType.DMA],
    )
    def k(x_hbm, pos_hbm, out_hbm, idx_v, rows_v, sem):
        wid = jax.lax.axis_index("s") * 2 + jax.lax.axis_index("c")
        base = wid * bpw
        trow = jax.lax.rem(base, T)
        pltpu.sync_copy(pos_hbm.at[pl.ds(base, bpw)], idx_v)
        pltpu.sync_copy(x_hbm.at[pl.ds(trow, bpw)], rows_v)
        pltpu.async_copy(rows_v, out_hbm.at[idx_v], sem).wait()

    return k(x, pos)


def _sc_combine(rows, idx0, idx1):
    T = idx0.shape[0]
    D = rows.shape[1]
    tpw = T // _NW
    CH = 32
    nch = tpw // CH

    @functools.partial(
        pl.kernel, mesh=_sc_mesh(),
        out_type=jax.ShapeDtypeStruct((T, D), rows.dtype),
        scratch_types=[pltpu.VMEM((CH,), jnp.int32),
                       pltpu.VMEM((CH,), jnp.int32),
                       pltpu.VMEM((CH, D), jnp.float32),
                       pltpu.VMEM((CH, D), jnp.float32),
                       pltpu.VMEM((CH, D), jnp.float32),
                       pltpu.SemaphoreType.DMA],
    )
    def k(rows_hbm, idx0_hbm, idx1_hbm, out_hbm, i0_v, i1_v, a_v, b_v, o_v, sem):
        wid = jax.lax.axis_index("s") * 2 + jax.lax.axis_index("c")
        for c in range(nch):
            base = wid * tpw + c * CH
            pltpu.sync_copy(idx0_hbm.at[pl.ds(base, CH)], i0_v)
            pltpu.sync_copy(idx1_hbm.at[pl.ds(base, CH)], i1_v)
            cp0 = pltpu.async_copy(rows_hbm.at[i0_v], a_v, sem)
            cp1 = pltpu.async_copy(rows_hbm.at[i1_v], b_v, sem)
            cp0.wait()
            cp1.wait()

            def body(r, _):
                for j in range(D // 16):
                    sl = pl.ds(16 * j, 16)
                    o_v[r, sl] = a_v[r, sl] + b_v[r, sl]
                return 0

            jax.lax.fori_loop(0, CH, body, 0)
            pltpu.sync_copy(o_v, out_hbm.at[pl.ds(base, CH)])

    return k(rows, idx0, idx1)


def _moe(x, lp):
    T = x.shape[0]
    P = TOPK * T
    NB = P // R_BLK
    G = NB + N_EXP - 1

    logits = x @ lp['router']
    pos, gflat, offs, eids, bids, valids = _routing_plan(logits)
    pos1 = pos.reshape(P)
    offs = offs.reshape(N_EXP + 1)
    eids = eids.reshape(G)
    bids = bids.reshape(G)
    valids = valids.reshape(G)

    gs = jnp.zeros((P, 1), x.dtype).at[pos1].set(gflat)
    xs = _sc_scatter_rows(x, pos1)

    o_pairs = pl.pallas_call(
        _gmm_body,
        grid_spec=pltpu.PrefetchScalarGridSpec(
            num_scalar_prefetch=4,
            grid=(G,),
            in_specs=[
                pl.BlockSpec((R_BLK, D_MODEL), lambda s, b, e, v, o: (b[s], 0)),
                pl.BlockSpec((R_BLK, 1), lambda s, b, e, v, o: (b[s], 0)),
                pl.BlockSpec((1, D_MODEL, D_FF), lambda s, b, e, v, o: (e[s], 0, 0)),
                pl.BlockSpec((1, 1, D_FF), lambda s, b, e, v, o: (e[s], 0, 0)),
                pl.BlockSpec((1, D_FF, D_MODEL), lambda s, b, e, v, o: (e[s], 0, 0)),
                pl.BlockSpec((1, 1, D_MODEL), lambda s, b, e, v, o: (e[s], 0, 0)),
            ],
            out_specs=pl.BlockSpec((R_BLK, D_MODEL), lambda s, b, e, v, o: (b[s], 0)),
        ),
        out_shape=jax.ShapeDtypeStruct((P, D_MODEL), x.dtype),
    )(bids, eids, valids, offs,
      xs, gs, lp['w1'], lp['b1'].reshape(N_EXP, 1, D_FF),
      lp['w2'], lp['b2'].reshape(N_EXP, 1, D_MODEL))

    return _sc_combine(o_pairs, pos1[:T], pos1[T:])


def _forward(particles, action, domain_id, params):
    a = _ln(action @ params['ap_w'] + params['ap_b'], params['ap_g'], params['ap_be'])
    x = particles + a[:, None, :]
    x = x + params['dom'][domain_id][:, None, :]
    for lp in params['layers']:
        xn = _ln(x, lp['g1'], lp['b1n'])
        x = x + _mha(xn, lp)
        xn = _ln(x, lp['g2'], lp['b2n'])
        Bq, T, D = x.shape
        x = x + _moe(xn.reshape(Bq * T, D), lp).reshape(Bq, T, D)
    out = _ln(x, params['out_g'], params['out_bn'])
    return out @ params['op_w'] + params['op_b']


def kernel(particles, action, domain_id, params):
    return _forward(particles, action, domain_id, params)


# speedup vs baseline: 1.0415x; 1.0415x over previous
import functools

import jax
import jax.numpy as jnp
from jax.experimental import pallas as pl
from jax.experimental.pallas import tpu as pltpu
from jax.experimental.pallas import tpu_sc as plsc

D_MODEL = 768
D_FF = 3072
N_EXP = 8
TOPK = 2
N_HEADS = 12
EPS = 1e-5
F_BLK = 768


def _ln(x, g, b):
    m = x.mean(-1, keepdims=True)
    v = ((x - m) ** 2).mean(-1, keepdims=True)
    return (x - m) / jnp.sqrt(v + EPS) * g + b


def _mha(x, lp):
    Bq, T, D = x.shape
    H = N_HEADS
    hd = D // H
    q = (x @ lp['wq'] + lp['bq']).reshape(Bq, T, H, hd).transpose(0, 2, 1, 3)
    k = (x @ lp['wk'] + lp['bk']).reshape(Bq, T, H, hd).transpose(0, 2, 1, 3)
    v = (x @ lp['wv'] + lp['bv']).reshape(Bq, T, H, hd).transpose(0, 2, 1, 3)
    s = jnp.einsum('bhtd,bhsd->bhts', q, k) / jnp.sqrt(jnp.float32(hd))
    a = jax.nn.softmax(s, axis=-1)
    o = jnp.einsum('bhts,bhsd->bhtd', a, v).transpose(0, 2, 1, 3).reshape(Bq, T, D)
    return o @ lp['wo'] + lp['bo']


R_BLK = 512


def _gmm_body(bids_ref, eids_ref, valids_ref, offs_ref,
              xs_ref, gs_ref, w1_ref, b1_ref, w2_ref, b2_ref, out_ref):
    s = pl.program_id(0)
    bid = bids_ref[s]
    eid = eids_ref[s]
    prev_bid = bids_ref[jnp.maximum(s - 1, 0)]
    first = (s == 0) | (bid != prev_bid)

    @pl.when(first)
    def _init():
        out_ref[...] = jnp.zeros_like(out_ref)

    @pl.when(valids_ref[s] > 0)
    def _compute():
        rows = bid * R_BLK + jax.lax.broadcasted_iota(jnp.int32, (R_BLK, 1), 0)
        mask = (rows >= offs_ref[eid]) & (rows < offs_ref[eid + 1])
        x = xs_ref[...]
        h = jnp.dot(x, w1_ref[0], preferred_element_type=jnp.float32)
        h = h + b1_ref[0, 0]
        h = 0.5 * h * (1.0 + jax.lax.erf(h * 0.7071067811865476))
        o = jnp.dot(h, w2_ref[0], preferred_element_type=jnp.float32)
        o = (o + b2_ref[0, 0]) * gs_ref[...]
        out_ref[...] += jnp.where(mask, o, 0.0)


def _routing_body(lg_ref, pos_ref, gf_ref, offs_ref, eids_ref, bids_ref, valids_ref):
    T = lg_ref.shape[0]
    P = TOPK * T
    NB = P // R_BLK
    G = NB + N_EXP - 1
    l = lg_ref[...]
    io8 = jax.lax.broadcasted_iota(jnp.int32, (T, N_EXP), 1)
    m1 = jnp.max(l, axis=1, keepdims=True)
    i1 = jnp.min(jnp.where(l == m1, io8, N_EXP), axis=1, keepdims=True)
    lm = jnp.where(io8 == i1, -jnp.inf, l)
    m2 = jnp.max(lm, axis=1, keepdims=True)
    i2 = jnp.min(jnp.where(lm == m2, io8, N_EXP), axis=1, keepdims=True)
    g0 = jax.nn.sigmoid(m1 - m2)
    gf_ref[...] = jnp.concatenate([g0, 1.0 - g0], axis=0)

    e_sel = jnp.concatenate([i1, i2], axis=0)
    iop8 = jax.lax.broadcasted_iota(jnp.int32, (P, N_EXP), 1)
    oh = (iop8 == e_sel).astype(jnp.float32)
    c = oh
    k = 1
    while k < P:
        c = c + jnp.concatenate(
            [jnp.zeros((k, N_EXP), jnp.float32), c[:-k]], axis=0)
        k *= 2
    counts = c[-1:]
    off_x = jax.lax.broadcasted_iota(jnp.int32, (N_EXP + 1, N_EXP), 0)
    off_e = jax.lax.broadcasted_iota(jnp.int32, (N_EXP + 1, N_EXP), 1)
    offs = jnp.sum(jnp.where(off_e < off_x, counts, 0.0), axis=1,
                   keepdims=True)
    offs_ref[...] = offs.astype(jnp.int32)
    t = counts
    k = 1
    while k < N_EXP:
        t = t + jnp.concatenate(
            [jnp.zeros((1, k), jnp.float32), t[:, :-k]], axis=1)
        k *= 2
    offs_row = t - counts
    pos = jnp.sum(oh * (offs_row + c - 1.0), axis=1, keepdims=True)
    pos_ref[...] = pos.astype(jnp.int32)

    offs_e_incl = offs_row + counts
    first = jnp.floor(offs_row / R_BLK)
    last = jnp.floor(jnp.maximum(offs_e_incl - 1.0, 0.0) / R_BLK)
    nvis = jnp.where(counts > 0, last - first + 1.0, 0.0)
    ioge = jax.lax.broadcasted_iota(jnp.int32, (G, N_EXP), 1)
    cumv = nvis
    k = 1
    while k < N_EXP:
        cumv = cumv + jnp.concatenate(
            [jnp.zeros((1, k), jnp.float32), cumv[:, :-k]], axis=1)
        k *= 2
    sg = jax.lax.broadcasted_iota(jnp.int32, (G, 1), 0).astype(jnp.float32)
    eids = jnp.sum((sg >= cumv).astype(jnp.int32), axis=1, keepdims=True)
    eids = jnp.clip(eids, 0, N_EXP - 1)
    valids = (sg < cumv[0, N_EXP - 1]).astype(jnp.int32)
    ohe = (ioge == eids).astype(jnp.float32)
    vstart = jnp.sum(ohe * (cumv - nvis), axis=1, keepdims=True)
    firsts = jnp.sum(ohe * first, axis=1, keepdims=True)
    bids = jnp.clip(firsts + sg - vstart, 0.0, NB - 1.0)
    eids_ref[...] = eids
    bids_ref[...] = bids.astype(jnp.int32)
    valids_ref[...] = valids


def _routing_plan(logits):
    T, E = logits.shape
    P = TOPK * T
    NB = P // R_BLK
    G = NB + N_EXP - 1
    i32 = jnp.int32
    return pl.pallas_call(
        _routing_body,
        out_shape=(jax.ShapeDtypeStruct((P, 1), i32),
                   jax.ShapeDtypeStruct((P, 1), jnp.float32),
                   jax.ShapeDtypeStruct((E + 1, 1), i32),
                   jax.ShapeDtypeStruct((G, 1), i32),
                   jax.ShapeDtypeStruct((G, 1), i32),
                   jax.ShapeDtypeStruct((G, 1), i32)),
    )(logits)


_NW = 32


def _sc_mesh():
    return plsc.VectorSubcoreMesh(core_axis_name="c", subcore_axis_name="s")


def _sc_gather(table, idx):
    B = idx.shape[0]
    D = table.shape[1]
    bpw = B // _NW

    @functools.partial(
        pl.kernel, mesh=_sc_mesh(),
        out_type=jax.ShapeDtypeStruct((B, D), table.dtype),
        scratch_types=[pltpu.VMEM((bpw,), jnp.int32),
                       pltpu.VMEM((bpw, D), jnp.float32),
                       pltpu.SemaphoreType.DMA],
    )
    def k(table_hbm, idx_hbm, out_hbm, idx_v, rows_v, sem):
        wid = jax.lax.axis_index("s") * 2 + jax.lax.axis_index("c")
        base = wid * bpw
        pltpu.sync_copy(idx_hbm.at[pl.ds(base, bpw)], idx_v)
        pltpu.async_copy(table_hbm.at[idx_v], rows_v, sem).wait()
        pltpu.sync_copy(rows_v, out_hbm.at[pl.ds(base, bpw)])

    return k(table, idx)


def _sc_scatter_rows(x, pos):
    T, D = x.shape
    P = pos.shape[0]
    bpw = P // _NW

    @functools.partial(
        pl.kernel, mesh=_sc_mesh(),
        out_type=jax.ShapeDtypeStruct((P, D), x.dtype),
        scratch_types=[pltpu.VMEM((bpw,), jnp.int32),
                       pltpu.VMEM((bpw, D), jnp.float32),
                       pltpu.SemaphoreType.DMA],
    )
    def k(x_hbm, pos_hbm, out_hbm, idx_v, rows_v, sem):
        wid = jax.lax.axis_index("s") * 2 + jax.lax.axis_index("c")
        base = wid * bpw
        trow = jax.lax.rem(base, T)
        pltpu.sync_copy(pos_hbm.at[pl.ds(base, bpw)], idx_v)
        pltpu.sync_copy(x_hbm.at[pl.ds(trow, bpw)], rows_v)
        pltpu.async_copy(rows_v, out_hbm.at[idx_v], sem).wait()

    return k(x, pos)


def _sc_combine(rows, idx0, idx1):
    T = idx0.shape[0]
    D = rows.shape[1]
    tpw = T // _NW
    CH = 32
    nch = tpw // CH

    @functools.partial(
        pl.kernel, mesh=_sc_mesh(),
        out_type=jax.ShapeDtypeStruct((T, D), rows.dtype),
        scratch_types=[pltpu.VMEM((CH,), jnp.int32),
                       pltpu.VMEM((CH,), jnp.int32),
                       pltpu.VMEM((CH, D), jnp.float32),
                       pltpu.VMEM((CH, D), jnp.float32),
                       pltpu.VMEM((CH, D), jnp.float32),
                       pltpu.SemaphoreType.DMA],
    )
    def k(rows_hbm, idx0_hbm, idx1_hbm, out_hbm, i0_v, i1_v, a_v, b_v, o_v, sem):
        wid = jax.lax.axis_index("s") * 2 + jax.lax.axis_index("c")
        for c in range(nch):
            base = wid * tpw + c * CH
            pltpu.sync_copy(idx0_hbm.at[pl.ds(base, CH)], i0_v)
            pltpu.sync_copy(idx1_hbm.at[pl.ds(base, CH)], i1_v)
            cp0 = pltpu.async_copy(rows_hbm.at[i0_v], a_v, sem)
            cp1 = pltpu.async_copy(rows_hbm.at[i1_v], b_v, sem)
            cp0.wait()
            cp1.wait()

            def body(r, _):
                for j in range(D // 16):
                    sl = pl.ds(16 * j, 16)
                    o_v[r, sl] = a_v[r, sl] + b_v[r, sl]
                return 0

            jax.lax.fori_loop(0, CH, body, 0)
            pltpu.sync_copy(o_v, out_hbm.at[pl.ds(base, CH)])

    return k(rows, idx0, idx1)


def _moe(x, lp):
    T = x.shape[0]
    P = TOPK * T
    NB = P // R_BLK
    G = NB + N_EXP - 1

    logits = x @ lp['router']
    pos, gflat, offs, eids, bids, valids = _routing_plan(logits)
    pos1 = pos.reshape(P)
    offs = offs.reshape(N_EXP + 1)
    eids = eids.reshape(G)
    bids = bids.reshape(G)
    valids = valids.reshape(G)

    gs = jnp.zeros((P, 1), x.dtype).at[pos1].set(gflat)
    xs = _sc_scatter_rows(x, pos1)

    o_pairs = pl.pallas_call(
        _gmm_body,
        grid_spec=pltpu.PrefetchScalarGridSpec(
            num_scalar_prefetch=4,
            grid=(G,),
            in_specs=[
                pl.BlockSpec((R_BLK, D_MODEL), lambda s, b, e, v, o: (b[s], 0)),
                pl.BlockSpec((R_BLK, 1), lambda s, b, e, v, o: (b[s], 0)),
                pl.BlockSpec((1, D_MODEL, D_FF), lambda s, b, e, v, o: (e[s], 0, 0)),
                pl.BlockSpec((1, 1, D_FF), lambda s, b, e, v, o: (e[s], 0, 0)),
                pl.BlockSpec((1, D_FF, D_MODEL), lambda s, b, e, v, o: (e[s], 0, 0)),
                pl.BlockSpec((1, 1, D_MODEL), lambda s, b, e, v, o: (e[s], 0, 0)),
            ],
            out_specs=pl.BlockSpec((R_BLK, D_MODEL), lambda s, b, e, v, o: (b[s], 0)),
        ),
        out_shape=jax.ShapeDtypeStruct((P, D_MODEL), x.dtype),
    )(bids, eids, valids, offs,
      xs, gs, lp['w1'], lp['b1'].reshape(N_EXP, 1, D_FF),
      lp['w2'], lp['b2'].reshape(N_EXP, 1, D_MODEL))

    return _sc_combine(o_pairs, pos1[:T], pos1[T:])


def _forward(particles, action, domain_id, params):
    a = _ln(action @ params['ap_w'] + params['ap_b'], params['ap_g'], params['ap_be'])
    x = particles + a[:, None, :]
    x = x + params['dom'][domain_id][:, None, :]
    for lp in params['layers']:
        xn = _ln(x, lp['g1'], lp['b1n'])
        x = x + _mha(xn, lp)
        xn = _ln(x, lp['g2'], lp['b2n'])
        Bq, T, D = x.shape
        x = x + _moe(xn.reshape(Bq * T, D), lp).reshape(Bq, T, D)
    out = _ln(x, params['out_g'], params['out_bn'])
    return out @ params['op_w'] + params['op_b']


def kernel(particles, action, domain_id, params):
    return _forward(particles, action, domain_id, params)
